# baseline (speedup 1.0000x reference)
import functools

import jax
import jax.numpy as jnp
from jax import lax
from jax.experimental import pallas as pl
from jax.experimental.pallas import tpu as pltpu
from jax.experimental.pallas import tpu_sc as plsc

N = 10000
FEAT = 128
E = 320000
NUM_CORES = 2
TILES = 16
LANES = 16
N_PAD = 10240
SLICE = N_PAD // TILES
UNROLL = 8
EGRP = LANES * UNROLL
COL_TILES = E // 128
HI_SUBCORES = 4
G_HI = 157
G_LO = 156
E_TILE_HI = G_HI * 128
E_TILE_LO = G_LO * 128
LAYERS = 3
ROW_BLK = 1024
OUT_GRID = (N + ROW_BLK - 1) // ROW_BLK
SUB = ROW_BLK // FEAT


def _sc_spmm3(feat_g, eidx_g, feat_p, eidx_p):
    mesh = plsc.VectorSubcoreMesh(
        core_axis_name="c", subcore_axis_name="s", num_cores=NUM_CORES)

    @functools.partial(
        pl.kernel,
        out_type=(jax.ShapeDtypeStruct((N_PAD,), jnp.float32),
                  jax.ShapeDtypeStruct((N_PAD,), jnp.float32)),
        mesh=mesh,
        scratch_types=[
            pltpu.VMEM((2, E_TILE_HI), jnp.int32),
            pltpu.VMEM((N_PAD,), jnp.float32),
            pltpu.VMEM((N_PAD,), jnp.float32),
            pltpu.VMEM((SLICE,), jnp.float32),
            pltpu.VMEM((TILES, SLICE), jnp.float32),
            pltpu.VMEM_SHARED((TILES, N_PAD), jnp.float32),
            pltpu.VMEM_SHARED((N_PAD,), jnp.float32),
        ],
        compiler_params=pltpu.CompilerParams(needs_layout_passes=False),
    )
    def spmm3(fg_hbm, eg_hbm, fp_hbm, ep_hbm,
              outg_hbm, outp_hbm,
              e_v, x_v, acc_v, red_v, tmp_v, stage, xshare):
        cid = lax.axis_index("c")
        sid = lax.axis_index("s")

        def zero_acc():
            zvec = jnp.zeros((LANES,), jnp.float32)

            def zloop(i, c):
                for u in range(8):
                    acc_v[pl.ds((i * 8 + u) * LANES, LANES)] = zvec
                return c

            lax.fori_loop(0, N_PAD // LANES // 8, zloop, 0)

        def run(feat_hbm, e_hbm, out_hbm):
            @pl.when(sid < HI_SUBCORES)
            def _():
                pltpu.sync_copy(
                    e_hbm.at[:, pl.ds(sid * E_TILE_HI, E_TILE_HI)], e_v)

            @pl.when(sid >= HI_SUBCORES)
            def _():
                start = (HI_SUBCORES * E_TILE_HI
                         + (sid - HI_SUBCORES) * E_TILE_LO)
                pltpu.sync_copy(e_hbm.at[:, pl.ds(start, E_TILE_LO)],
                                e_v.at[:, pl.ds(0, E_TILE_LO)])

            eiters = jnp.where(sid < HI_SUBCORES, G_HI, G_LO)
            zero_acc()
            pltpu.sync_copy(feat_hbm, x_v.at[pl.ds(0, N)])
            myoff = sid * SLICE

            for layer in range(LAYERS):
                def eloop(i, c):
                    gbase = i * EGRP
                    for u in range(UNROLL):
                        off = gbase + u * LANES
                        ci = e_v[1, pl.ds(off, LANES)]
                        ri = e_v[0, pl.ds(off, LANES)]
                        vals = plsc.load_gather(x_v, [ci])
                        plsc.addupdate_scatter(acc_v, [ri], vals)
                    return c

                lax.fori_loop(0, eiters, eloop, 0)

                pltpu.sync_copy(acc_v, stage.at[sid])
                if layer < LAYERS - 1:
                    zero_acc()
                plsc.subcore_barrier()

                pltpu.sync_copy(stage.at[:, pl.ds(myoff, SLICE)], tmp_v)

                def rloop(j, c):
                    off = j * LANES
                    s = tmp_v[0, pl.ds(off, LANES)]
                    for t in range(1, TILES):
                        s = s + tmp_v[t, pl.ds(off, LANES)]
                    red_v[pl.ds(off, LANES)] = s
                    return c

                lax.fori_loop(0, SLICE // LANES, rloop, 0)

                if layer < LAYERS - 1:
                    pltpu.sync_copy(red_v, xshare.at[pl.ds(myoff, SLICE)])
                    plsc.subcore_barrier()
                    pltpu.sync_copy(xshare, x_v)
                else:
                    pltpu.sync_copy(red_v, out_hbm.at[pl.ds(myoff, SLICE)])

        @pl.when(cid == 0)
        def _():
            run(fg_hbm, eg_hbm, outg_hbm)

        @pl.when(cid == 1)
        def _():
            run(fp_hbm, ep_hbm, outp_hbm)

    return spmm3(feat_g, eidx_g, feat_p, eidx_p)


def _tc_wc(W0, W1, W2):
    def body(w0_ref, w1_ref, w2_ref, wc_ref):
        w01 = jnp.dot(w0_ref[...], w1_ref[...],
                      preferred_element_type=jnp.float32,
                      precision=lax.Precision.HIGHEST)
        wc_ref[...] = jnp.dot(w01, w2_ref[...],
                              preferred_element_type=jnp.float32,
                              precision=lax.Precision.HIGHEST)

    return pl.pallas_call(
        body,
        out_shape=jax.ShapeDtypeStruct((1, FEAT), jnp.float32),
    )(W0, W1, W2)


def _tc_outer(sg2d, sp2d, wc):
    contract = (((0,), (0,)), ((), ()))

    def body(sg_ref, sp_ref, wc_ref, og_ref, op_ref):
        w = wc_ref[...]
        for q in range(SUB):
            og_ref[q * FEAT:(q + 1) * FEAT, :] = lax.dot_general(
                sg_ref[q:q + 1, :], w, contract,
                preferred_element_type=jnp.float32)
            op_ref[q * FEAT:(q + 1) * FEAT, :] = lax.dot_general(
                sp_ref[q:q + 1, :], w, contract,
                preferred_element_type=jnp.float32)

    return pl.pallas_call(
        body,
        grid=(OUT_GRID,),
        in_specs=[
            pl.BlockSpec((SUB, FEAT), lambda i: (i, 0)),
            pl.BlockSpec((SUB, FEAT), lambda i: (i, 0)),
            pl.BlockSpec((1, FEAT), lambda i: (0, 0)),
        ],
        out_specs=[
            pl.BlockSpec((ROW_BLK, FEAT), lambda i: (i, 0)),
            pl.BlockSpec((ROW_BLK, FEAT), lambda i: (i, 0)),
        ],
        out_shape=(jax.ShapeDtypeStruct((N, FEAT), jnp.float32),
                   jax.ShapeDtypeStruct((N, FEAT), jnp.float32)),
        compiler_params=pltpu.CompilerParams(
            dimension_semantics=("parallel",)),
    )(sg2d, sp2d, wc)


def kernel(feat_gene, edge_index_gene, feat_pro, edge_index_pro, W0, W1, W2):
    fg = feat_gene.reshape(N)
    fp = feat_pro.reshape(N)
    wc = _tc_wc(W0, W1, W2)
    sg, sp = _sc_spmm3(fg, edge_index_gene, fp, edge_index_pro)
    return _tc_outer(sg.reshape(N_PAD // FEAT, FEAT),
                     sp.reshape(N_PAD // FEAT, FEAT), wc)


# speedup vs baseline: 1.4036x; 1.4036x over previous
import functools

import jax
import jax.numpy as jnp
from jax import lax
from jax.experimental import pallas as pl
from jax.experimental.pallas import tpu as pltpu
from jax.experimental.pallas import tpu_sc as plsc

N = 10000
FEAT = 128
E = 320000
NUM_CORES = 2
TILES = 16
LANES = 16
N_PAD = 10240
SLICE = N_PAD // TILES
UNROLL = 8
BATCH = 4
EGRP = LANES * UNROLL
COL_TILES = E // 128
HI_SUBCORES = 4
G_HI = 157
G_LO = 156
E_TILE_HI = G_HI * 128
E_TILE_LO = G_LO * 128
LAYERS = 3
ROW_BLK = 1024
OUT_GRID = (N + ROW_BLK - 1) // ROW_BLK
SUB = ROW_BLK // FEAT


def _sc_spmm3(feat_g, eidx_g, feat_p, eidx_p):
    mesh = plsc.VectorSubcoreMesh(
        core_axis_name="c", subcore_axis_name="s", num_cores=NUM_CORES)

    @functools.partial(
        pl.kernel,
        out_type=(jax.ShapeDtypeStruct((N_PAD,), jnp.float32),
                  jax.ShapeDtypeStruct((N_PAD,), jnp.float32)),
        mesh=mesh,
        scratch_types=[
            pltpu.VMEM((2, E_TILE_HI), jnp.int32),
            pltpu.VMEM((N_PAD,), jnp.float32),
            pltpu.VMEM((N_PAD,), jnp.float32),
            pltpu.VMEM((SLICE,), jnp.float32),
            pltpu.VMEM((TILES, SLICE), jnp.float32),
            pltpu.VMEM_SHARED((TILES, N_PAD), jnp.float32),
            pltpu.VMEM_SHARED((N_PAD,), jnp.float32),
        ],
        compiler_params=pltpu.CompilerParams(needs_layout_passes=False),
    )
    def spmm3(fg_hbm, eg_hbm, fp_hbm, ep_hbm,
              outg_hbm, outp_hbm,
              e_v, x_v, acc_v, red_v, tmp_v, stage, xshare):
        cid = lax.axis_index("c")
        sid = lax.axis_index("s")

        def zero_acc():
            zvec = jnp.zeros((LANES,), jnp.float32)

            def zloop(i, c):
                for u in range(8):
                    acc_v[pl.ds((i * 8 + u) * LANES, LANES)] = zvec
                return c

            lax.fori_loop(0, N_PAD // LANES // 8, zloop, 0)

        def run(feat_hbm, e_hbm, out_hbm):
            @pl.when(sid < HI_SUBCORES)
            def _():
                pltpu.sync_copy(
                    e_hbm.at[:, pl.ds(sid * E_TILE_HI, E_TILE_HI)], e_v)

            @pl.when(sid >= HI_SUBCORES)
            def _():
                start = (HI_SUBCORES * E_TILE_HI
                         + (sid - HI_SUBCORES) * E_TILE_LO)
                pltpu.sync_copy(e_hbm.at[:, pl.ds(start, E_TILE_LO)],
                                e_v.at[:, pl.ds(0, E_TILE_LO)])

            eiters = jnp.where(sid < HI_SUBCORES, G_HI, G_LO)
            zero_acc()
            pltpu.sync_copy(feat_hbm, x_v.at[pl.ds(0, N)])
            myoff = sid * SLICE

            for layer in range(LAYERS):
                def eloop(i, c):
                    gbase = i * EGRP
                    for b in range(0, UNROLL, BATCH):
                        offs = [gbase + (b + k) * LANES
                                for k in range(BATCH)]
                        cis = [e_v[1, pl.ds(o, LANES)] for o in offs]
                        ris = [e_v[0, pl.ds(o, LANES)] for o in offs]
                        vals = [plsc.load_gather(x_v, [ci]) for ci in cis]
                        for k in range(BATCH):
                            plsc.addupdate_scatter(acc_v, [ris[k]], vals[k])
                    return c

                lax.fori_loop(0, eiters, eloop, 0)

                pltpu.sync_copy(acc_v, stage.at[sid])
                if layer < LAYERS - 1:
                    zero_acc()
                plsc.subcore_barrier()

                pltpu.sync_copy(stage.at[:, pl.ds(myoff, SLICE)], tmp_v)

                def rloop(j, c):
                    off = j * LANES
                    s = tmp_v[0, pl.ds(off, LANES)]
                    for t in range(1, TILES):
                        s = s + tmp_v[t, pl.ds(off, LANES)]
                    red_v[pl.ds(off, LANES)] = s
                    return c

                lax.fori_loop(0, SLICE // LANES, rloop, 0)

                if layer < LAYERS - 1:
                    pltpu.sync_copy(red_v, xshare.at[pl.ds(myoff, SLICE)])
                    plsc.subcore_barrier()
                    pltpu.sync_copy(xshare, x_v)
                else:
                    pltpu.sync_copy(red_v, out_hbm.at[pl.ds(myoff, SLICE)])

        @pl.when(cid == 0)
        def _():
            run(fg_hbm, eg_hbm, outg_hbm)

        @pl.when(cid == 1)
        def _():
            run(fp_hbm, ep_hbm, outp_hbm)

    return spmm3(feat_g, eidx_g, feat_p, eidx_p)


def _tc_wc(W0, W1, W2):
    def body(w0_ref, w1_ref, w2_ref, wc_ref):
        w01 = jnp.dot(w0_ref[...], w1_ref[...],
                      preferred_element_type=jnp.float32,
                      precision=lax.Precision.HIGHEST)
        wc_ref[...] = jnp.dot(w01, w2_ref[...],
                              preferred_element_type=jnp.float32,
                              precision=lax.Precision.HIGHEST)

    return pl.pallas_call(
        body,
        out_shape=jax.ShapeDtypeStruct((1, FEAT), jnp.float32),
    )(W0, W1, W2)


def _tc_outer(sg2d, sp2d, wc):
    contract = (((0,), (0,)), ((), ()))

    def body(sg_ref, sp_ref, wc_ref, og_ref, op_ref):
        w = wc_ref[...]
        for q in range(SUB):
            og_ref[q * FEAT:(q + 1) * FEAT, :] = lax.dot_general(
                sg_ref[q:q + 1, :], w, contract,
                preferred_element_type=jnp.float32)
            op_ref[q * FEAT:(q + 1) * FEAT, :] = lax.dot_general(
                sp_ref[q:q + 1, :], w, contract,
                preferred_element_type=jnp.float32)

    return pl.pallas_call(
        body,
        grid=(OUT_GRID,),
        in_specs=[
            pl.BlockSpec((SUB, FEAT), lambda i: (i, 0)),
            pl.BlockSpec((SUB, FEAT), lambda i: (i, 0)),
            pl.BlockSpec((1, FEAT), lambda i: (0, 0)),
        ],
        out_specs=[
            pl.BlockSpec((ROW_BLK, FEAT), lambda i: (i, 0)),
            pl.BlockSpec((ROW_BLK, FEAT), lambda i: (i, 0)),
        ],
        out_shape=(jax.ShapeDtypeStruct((N, FEAT), jnp.float32),
                   jax.ShapeDtypeStruct((N, FEAT), jnp.float32)),
        compiler_params=pltpu.CompilerParams(
            dimension_semantics=("parallel",)),
    )(sg2d, sp2d, wc)


def kernel(feat_gene, edge_index_gene, feat_pro, edge_index_pro, W0, W1, W2):
    fg = feat_gene.reshape(N)
    fp = feat_pro.reshape(N)
    wc = _tc_wc(W0, W1, W2)
    sg, sp = _sc_spmm3(fg, edge_index_gene, fp, edge_index_pro)
    return _tc_outer(sg.reshape(N_PAD // FEAT, FEAT),
                     sp.reshape(N_PAD // FEAT, FEAT), wc)


# speedup vs baseline: 1.4424x; 1.0276x over previous
import functools

import jax
import jax.numpy as jnp
from jax import lax
from jax.experimental import pallas as pl
from jax.experimental.pallas import tpu as pltpu
from jax.experimental.pallas import tpu_sc as plsc

N = 10000
FEAT = 128
E = 320000
NUM_CORES = 2
TILES = 16
LANES = 16
N_PAD = 10240
SLICE = N_PAD // TILES
UNROLL = 8
BATCH = 8
EGRP = LANES * UNROLL
COL_TILES = E // 128
HI_SUBCORES = 4
G_HI = 157
G_LO = 156
E_TILE_HI = G_HI * 128
E_TILE_LO = G_LO * 128
LAYERS = 3
ROW_BLK = 1024
OUT_GRID = (N + ROW_BLK - 1) // ROW_BLK
SUB = ROW_BLK // FEAT


def _sc_spmm3(feat_g, eidx_g, feat_p, eidx_p):
    mesh = plsc.VectorSubcoreMesh(
        core_axis_name="c", subcore_axis_name="s", num_cores=NUM_CORES)

    @functools.partial(
        pl.kernel,
        out_type=(jax.ShapeDtypeStruct((N_PAD,), jnp.float32),
                  jax.ShapeDtypeStruct((N_PAD,), jnp.float32)),
        mesh=mesh,
        scratch_types=[
            pltpu.VMEM((2, E_TILE_HI), jnp.int32),
            pltpu.VMEM((N_PAD,), jnp.float32),
            pltpu.VMEM((N_PAD,), jnp.float32),
            pltpu.VMEM((SLICE,), jnp.float32),
            pltpu.VMEM((TILES, SLICE), jnp.float32),
            pltpu.VMEM_SHARED((TILES, N_PAD), jnp.float32),
            pltpu.VMEM_SHARED((N_PAD,), jnp.float32),
            pltpu.SemaphoreType.DMA,
        ],
        compiler_params=pltpu.CompilerParams(needs_layout_passes=False),
    )
    def spmm3(fg_hbm, eg_hbm, fp_hbm, ep_hbm,
              outg_hbm, outp_hbm,
              e_v, x_v, acc_v, red_v, tmp_v, stage, xshare, sem):
        cid = lax.axis_index("c")
        sid = lax.axis_index("s")

        def zero_acc():
            zvec = jnp.zeros((LANES,), jnp.float32)

            def zloop(i, c):
                for u in range(8):
                    acc_v[pl.ds((i * 8 + u) * LANES, LANES)] = zvec
                return c

            lax.fori_loop(0, N_PAD // LANES // 8, zloop, 0)

        def run(feat_hbm, e_hbm, out_hbm):
            @pl.when(sid < HI_SUBCORES)
            def _():
                pltpu.sync_copy(
                    e_hbm.at[:, pl.ds(sid * E_TILE_HI, E_TILE_HI)], e_v)

            @pl.when(sid >= HI_SUBCORES)
            def _():
                start = (HI_SUBCORES * E_TILE_HI
                         + (sid - HI_SUBCORES) * E_TILE_LO)
                pltpu.sync_copy(e_hbm.at[:, pl.ds(start, E_TILE_LO)],
                                e_v.at[:, pl.ds(0, E_TILE_LO)])

            eiters = jnp.where(sid < HI_SUBCORES, G_HI, G_LO)
            zero_acc()
            pltpu.sync_copy(feat_hbm, x_v.at[pl.ds(0, N)])
            myoff = sid * SLICE

            for layer in range(LAYERS):
                def eloop(i, c):
                    gbase = i * EGRP
                    for b in range(0, UNROLL, BATCH):
                        offs = [gbase + (b + k) * LANES
                                for k in range(BATCH)]
                        cis = [e_v[1, pl.ds(o, LANES)] for o in offs]
                        ris = [e_v[0, pl.ds(o, LANES)] for o in offs]
                        vals = [plsc.load_gather(x_v, [ci]) for ci in cis]
                        for k in range(BATCH):
                            plsc.addupdate_scatter(acc_v, [ris[k]], vals[k])
                    return c

                lax.fori_loop(0, eiters, eloop, 0)

                pltpu.sync_copy(acc_v, stage.at[sid])
                plsc.subcore_barrier()

                cp = pltpu.async_copy(
                    stage.at[:, pl.ds(myoff, SLICE)], tmp_v, sem)
                if layer < LAYERS - 1:
                    zero_acc()
                cp.wait()

                def rloop(j, c):
                    off = j * LANES
                    s = tmp_v[0, pl.ds(off, LANES)]
                    for t in range(1, TILES):
                        s = s + tmp_v[t, pl.ds(off, LANES)]
                    red_v[pl.ds(off, LANES)] = s
                    return c

                lax.fori_loop(0, SLICE // LANES, rloop, 0)

                if layer < LAYERS - 1:
                    pltpu.sync_copy(red_v, xshare.at[pl.ds(myoff, SLICE)])
                    plsc.subcore_barrier()
                    pltpu.sync_copy(xshare, x_v)
                else:
                    pltpu.sync_copy(red_v, out_hbm.at[pl.ds(myoff, SLICE)])

        @pl.when(cid == 0)
        def _():
            run(fg_hbm, eg_hbm, outg_hbm)

        @pl.when(cid == 1)
        def _():
            run(fp_hbm, ep_hbm, outp_hbm)

    return spmm3(feat_g, eidx_g, feat_p, eidx_p)


def _tc_wc(W0, W1, W2):
    def body(w0_ref, w1_ref, w2_ref, wc_ref):
        w01 = jnp.dot(w0_ref[...], w1_ref[...],
                      preferred_element_type=jnp.float32,
                      precision=lax.Precision.HIGHEST)
        wc_ref[...] = jnp.dot(w01, w2_ref[...],
                              preferred_element_type=jnp.float32,
                              precision=lax.Precision.HIGHEST)

    return pl.pallas_call(
        body,
        out_shape=jax.ShapeDtypeStruct((1, FEAT), jnp.float32),
    )(W0, W1, W2)


def _tc_outer(sg2d, sp2d, wc):
    contract = (((0,), (0,)), ((), ()))

    def body(sg_ref, sp_ref, wc_ref, og_ref, op_ref):
        w = wc_ref[...]
        for q in range(SUB):
            og_ref[q * FEAT:(q + 1) * FEAT, :] = lax.dot_general(
                sg_ref[q:q + 1, :], w, contract,
                preferred_element_type=jnp.float32)
            op_ref[q * FEAT:(q + 1) * FEAT, :] = lax.dot_general(
                sp_ref[q:q + 1, :], w, contract,
                preferred_element_type=jnp.float32)

    return pl.pallas_call(
        body,
        grid=(OUT_GRID,),
        in_specs=[
            pl.BlockSpec((SUB, FEAT), lambda i: (i, 0)),
            pl.BlockSpec((SUB, FEAT), lambda i: (i, 0)),
            pl.BlockSpec((1, FEAT), lambda i: (0, 0)),
        ],
        out_specs=[
            pl.BlockSpec((ROW_BLK, FEAT), lambda i: (i, 0)),
            pl.BlockSpec((ROW_BLK, FEAT), lambda i: (i, 0)),
        ],
        out_shape=(jax.ShapeDtypeStruct((N, FEAT), jnp.float32),
                   jax.ShapeDtypeStruct((N, FEAT), jnp.float32)),
        compiler_params=pltpu.CompilerParams(
            dimension_semantics=("parallel",)),
    )(sg2d, sp2d, wc)


def kernel(feat_gene, edge_index_gene, feat_pro, edge_index_pro, W0, W1, W2):
    fg = feat_gene.reshape(N)
    fp = feat_pro.reshape(N)
    wc = _tc_wc(W0, W1, W2)
    sg, sp = _sc_spmm3(fg, edge_index_gene, fp, edge_index_pro)
    return _tc_outer(sg.reshape(N_PAD // FEAT, FEAT),
                     sp.reshape(N_PAD // FEAT, FEAT), wc)


# speedup vs baseline: 1.5056x; 1.0438x over previous
import functools

import jax
import jax.numpy as jnp
from jax import lax
from jax.experimental import pallas as pl
from jax.experimental.pallas import tpu as pltpu
from jax.experimental.pallas import tpu_sc as plsc

N = 10000
FEAT = 128
E = 320000
NUM_CORES = 2
TILES = 16
LANES = 16
N_PAD = 10240
SLICE = N_PAD // TILES
UNROLL = 8
BATCH = 8
EGRP = LANES * UNROLL
COL_TILES = E // 128
HI_SUBCORES = 4
G_HI = 157
G_LO = 156
E_TILE_HI = G_HI * 128
E_TILE_LO = G_LO * 128
LAYERS = 3
ROW_BLK = 2048
OUT_GRID = (N + ROW_BLK - 1) // ROW_BLK
SUB = ROW_BLK // FEAT


def _sc_spmm3(feat_g, eidx_g, feat_p, eidx_p):
    mesh = plsc.VectorSubcoreMesh(
        core_axis_name="c", subcore_axis_name="s", num_cores=NUM_CORES)

    @functools.partial(
        pl.kernel,
        out_type=(jax.ShapeDtypeStruct((N_PAD,), jnp.float32),
                  jax.ShapeDtypeStruct((N_PAD,), jnp.float32)),
        mesh=mesh,
        scratch_types=[
            pltpu.VMEM((2, E_TILE_HI), jnp.int32),
            pltpu.VMEM((N_PAD,), jnp.float32),
            pltpu.VMEM((N_PAD,), jnp.float32),
            pltpu.VMEM((SLICE,), jnp.float32),
            pltpu.VMEM((TILES, SLICE), jnp.float32),
            pltpu.VMEM_SHARED((TILES, N_PAD), jnp.float32),
            pltpu.VMEM_SHARED((N_PAD,), jnp.float32),
            pltpu.SemaphoreType.DMA,
        ],
        compiler_params=pltpu.CompilerParams(needs_layout_passes=False),
    )
    def spmm3(fg_hbm, eg_hbm, fp_hbm, ep_hbm,
              outg_hbm, outp_hbm,
              e_v, x_v, acc_v, red_v, tmp_v, stage, xshare, sem):
        cid = lax.axis_index("c")
        sid = lax.axis_index("s")

        def zero_acc():
            zvec = jnp.zeros((LANES,), jnp.float32)

            def zloop(i, c):
                for u in range(8):
                    acc_v[pl.ds((i * 8 + u) * LANES, LANES)] = zvec
                return c

            lax.fori_loop(0, N_PAD // LANES // 8, zloop, 0)

        def run(feat_hbm, e_hbm, out_hbm):
            @pl.when(sid < HI_SUBCORES)
            def _():
                pltpu.sync_copy(
                    e_hbm.at[:, pl.ds(sid * E_TILE_HI, E_TILE_HI)], e_v)

            @pl.when(sid >= HI_SUBCORES)
            def _():
                start = (HI_SUBCORES * E_TILE_HI
                         + (sid - HI_SUBCORES) * E_TILE_LO)
                pltpu.sync_copy(e_hbm.at[:, pl.ds(start, E_TILE_LO)],
                                e_v.at[:, pl.ds(0, E_TILE_LO)])

            eiters = jnp.where(sid < HI_SUBCORES, G_HI, G_LO)
            zero_acc()
            pltpu.sync_copy(feat_hbm, x_v.at[pl.ds(0, N)])
            myoff = sid * SLICE

            for layer in range(LAYERS):
                def eloop(i, c):
                    gbase = i * EGRP
                    for b in range(0, UNROLL, BATCH):
                        offs = [gbase + (b + k) * LANES
                                for k in range(BATCH)]
                        cis = [e_v[1, pl.ds(o, LANES)] for o in offs]
                        ris = [e_v[0, pl.ds(o, LANES)] for o in offs]
                        vals = [plsc.load_gather(x_v, [ci]) for ci in cis]
                        for k in range(BATCH):
                            plsc.addupdate_scatter(acc_v, [ris[k]], vals[k])
                    return c

                lax.fori_loop(0, eiters, eloop, 0)

                pltpu.sync_copy(acc_v, stage.at[sid])
                plsc.subcore_barrier()

                cp = pltpu.async_copy(
                    stage.at[:, pl.ds(myoff, SLICE)], tmp_v, sem)
                if layer < LAYERS - 1:
                    zero_acc()
                cp.wait()

                def rloop(j, c):
                    off = j * LANES
                    s = tmp_v[0, pl.ds(off, LANES)]
                    for t in range(1, TILES):
                        s = s + tmp_v[t, pl.ds(off, LANES)]
                    red_v[pl.ds(off, LANES)] = s
                    return c

                lax.fori_loop(0, SLICE // LANES, rloop, 0)

                if layer < LAYERS - 1:
                    pltpu.sync_copy(red_v, xshare.at[pl.ds(myoff, SLICE)])
                    plsc.subcore_barrier()
                    pltpu.sync_copy(xshare, x_v)
                else:
                    pltpu.sync_copy(red_v, out_hbm.at[pl.ds(myoff, SLICE)])

        @pl.when(cid == 0)
        def _():
            run(fg_hbm, eg_hbm, outg_hbm)

        @pl.when(cid == 1)
        def _():
            run(fp_hbm, ep_hbm, outp_hbm)

    return spmm3(feat_g, eidx_g, feat_p, eidx_p)


def _tc_wc(W0, W1, W2):
    def body(w0_ref, w1_ref, w2_ref, wc_ref):
        w01 = jnp.dot(w0_ref[...], w1_ref[...],
                      preferred_element_type=jnp.float32,
                      precision=lax.Precision.HIGHEST)
        wc_ref[...] = jnp.dot(w01, w2_ref[...],
                              preferred_element_type=jnp.float32,
                              precision=lax.Precision.HIGHEST)

    return pl.pallas_call(
        body,
        out_shape=jax.ShapeDtypeStruct((1, FEAT), jnp.float32),
    )(W0, W1, W2)


def _tc_outer(sg2d, sp2d, wc):
    contract = (((0,), (0,)), ((), ()))

    def body(sg_ref, sp_ref, wc_ref, og_ref, op_ref):
        w = wc_ref[...]
        for q in range(SUB):
            og_ref[q * FEAT:(q + 1) * FEAT, :] = lax.dot_general(
                sg_ref[q:q + 1, :], w, contract,
                preferred_element_type=jnp.float32)
            op_ref[q * FEAT:(q + 1) * FEAT, :] = lax.dot_general(
                sp_ref[q:q + 1, :], w, contract,
                preferred_element_type=jnp.float32)

    return pl.pallas_call(
        body,
        grid=(OUT_GRID,),
        in_specs=[
            pl.BlockSpec((SUB, FEAT), lambda i: (i, 0)),
            pl.BlockSpec((SUB, FEAT), lambda i: (i, 0)),
            pl.BlockSpec((1, FEAT), lambda i: (0, 0)),
        ],
        out_specs=[
            pl.BlockSpec((ROW_BLK, FEAT), lambda i: (i, 0)),
            pl.BlockSpec((ROW_BLK, FEAT), lambda i: (i, 0)),
        ],
        out_shape=(jax.ShapeDtypeStruct((N, FEAT), jnp.float32),
                   jax.ShapeDtypeStruct((N, FEAT), jnp.float32)),
        compiler_params=pltpu.CompilerParams(
            dimension_semantics=("parallel",)),
    )(sg2d, sp2d, wc)


def kernel(feat_gene, edge_index_gene, feat_pro, edge_index_pro, W0, W1, W2):
    fg = feat_gene.reshape(N)
    fp = feat_pro.reshape(N)
    wc = _tc_wc(W0, W1, W2)
    sg, sp = _sc_spmm3(fg, edge_index_gene, fp, edge_index_pro)
    return _tc_outer(sg.reshape(N_PAD // FEAT, FEAT),
                     sp.reshape(N_PAD // FEAT, FEAT), wc)


# speedup vs baseline: 1.5472x; 1.0276x over previous
import functools

import jax
import jax.numpy as jnp
from jax import lax
from jax.experimental import pallas as pl
from jax.experimental.pallas import tpu as pltpu
from jax.experimental.pallas import tpu_sc as plsc

N = 10000
FEAT = 128
E = 320000
NUM_CORES = 2
TILES = 16
LANES = 16
N_PAD = 10240
SLICE = N_PAD // TILES
UNROLL = 8
BATCH = 8
EGRP = LANES * UNROLL
COL_TILES = E // 128
HI_SUBCORES = 4
G_HI = 157
G_LO = 156
E_TILE_HI = G_HI * 128
E_TILE_LO = G_LO * 128
LAYERS = 3
ROW_BLK = 5120
OUT_GRID = (N + ROW_BLK - 1) // ROW_BLK
SUB = ROW_BLK // FEAT


def _sc_spmm3(feat_g, eidx_g, feat_p, eidx_p):
    mesh = plsc.VectorSubcoreMesh(
        core_axis_name="c", subcore_axis_name="s", num_cores=NUM_CORES)

    @functools.partial(
        pl.kernel,
        out_type=(jax.ShapeDtypeStruct((N_PAD,), jnp.float32),
                  jax.ShapeDtypeStruct((N_PAD,), jnp.float32)),
        mesh=mesh,
        scratch_types=[
            pltpu.VMEM((2, E_TILE_HI), jnp.int32),
            pltpu.VMEM((N_PAD,), jnp.float32),
            pltpu.VMEM((N_PAD,), jnp.float32),
            pltpu.VMEM((SLICE,), jnp.float32),
            pltpu.VMEM((TILES, SLICE), jnp.float32),
            pltpu.VMEM_SHARED((TILES, N_PAD), jnp.float32),
            pltpu.VMEM_SHARED((N_PAD,), jnp.float32),
            pltpu.SemaphoreType.DMA,
        ],
        compiler_params=pltpu.CompilerParams(needs_layout_passes=False),
    )
    def spmm3(fg_hbm, eg_hbm, fp_hbm, ep_hbm,
              outg_hbm, outp_hbm,
              e_v, x_v, acc_v, red_v, tmp_v, stage, xshare, sem):
        cid = lax.axis_index("c")
        sid = lax.axis_index("s")

        def zero_acc():
            zvec = jnp.zeros((LANES,), jnp.float32)

            def zloop(i, c):
                for u in range(8):
                    acc_v[pl.ds((i * 8 + u) * LANES, LANES)] = zvec
                return c

            lax.fori_loop(0, N_PAD // LANES // 8, zloop, 0)

        def run(feat_hbm, e_hbm, out_hbm):
            estart = jnp.where(
                sid < HI_SUBCORES, sid * E_TILE_HI,
                HI_SUBCORES * E_TILE_HI + (sid - HI_SUBCORES) * E_TILE_LO)
            ecp = pltpu.async_copy(
                e_hbm.at[:, pl.ds(estart, E_TILE_LO)],
                e_v.at[:, pl.ds(0, E_TILE_LO)], sem)

            @pl.when(sid < HI_SUBCORES)
            def _():
                pltpu.sync_copy(
                    e_hbm.at[:, pl.ds(sid * E_TILE_HI + E_TILE_LO,
                                      E_TILE_HI - E_TILE_LO)],
                    e_v.at[:, pl.ds(E_TILE_LO, E_TILE_HI - E_TILE_LO)])

            eiters = jnp.where(sid < HI_SUBCORES, G_HI, G_LO)
            zero_acc()
            pltpu.sync_copy(feat_hbm, x_v.at[pl.ds(0, N)])
            ecp.wait()
            myoff = sid * SLICE

            for layer in range(LAYERS):
                def eloop(i, c):
                    gbase = i * EGRP
                    for b in range(0, UNROLL, BATCH):
                        offs = [gbase + (b + k) * LANES
                                for k in range(BATCH)]
                        cis = [e_v[1, pl.ds(o, LANES)] for o in offs]
                        ris = [e_v[0, pl.ds(o, LANES)] for o in offs]
                        vals = [plsc.load_gather(x_v, [ci]) for ci in cis]
                        for k in range(BATCH):
                            plsc.addupdate_scatter(acc_v, [ris[k]], vals[k])
                    return c

                lax.fori_loop(0, eiters, eloop, 0)

                pltpu.sync_copy(acc_v, stage.at[sid])
                plsc.subcore_barrier()

                cp = pltpu.async_copy(
                    stage.at[:, pl.ds(myoff, SLICE)], tmp_v, sem)
                if layer < LAYERS - 1:
                    zero_acc()
                cp.wait()

                def rloop(j, c):
                    off = j * LANES
                    s = tmp_v[0, pl.ds(off, LANES)]
                    for t in range(1, TILES):
                        s = s + tmp_v[t, pl.ds(off, LANES)]
                    red_v[pl.ds(off, LANES)] = s
                    return c

                lax.fori_loop(0, SLICE // LANES, rloop, 0)

                if layer < LAYERS - 1:
                    pltpu.sync_copy(red_v, xshare.at[pl.ds(myoff, SLICE)])
                    plsc.subcore_barrier()
                    pltpu.sync_copy(xshare, x_v)
                else:
                    pltpu.sync_copy(red_v, out_hbm.at[pl.ds(myoff, SLICE)])

        @pl.when(cid == 0)
        def _():
            run(fg_hbm, eg_hbm, outg_hbm)

        @pl.when(cid == 1)
        def _():
            run(fp_hbm, ep_hbm, outp_hbm)

    return spmm3(feat_g, eidx_g, feat_p, eidx_p)


def _tc_wc(W0, W1, W2):
    def body(w0_ref, w1_ref, w2_ref, wc_ref):
        w01 = jnp.dot(w0_ref[...], w1_ref[...],
                      preferred_element_type=jnp.float32,
                      precision=lax.Precision.HIGHEST)
        wc_ref[...] = jnp.dot(w01, w2_ref[...],
                              preferred_element_type=jnp.float32,
                              precision=lax.Precision.HIGHEST)

    return pl.pallas_call(
        body,
        out_shape=jax.ShapeDtypeStruct((1, FEAT), jnp.float32),
    )(W0, W1, W2)


def _tc_outer(sg2d, sp2d, wc):
    contract = (((0,), (0,)), ((), ()))

    def body(sg_ref, sp_ref, wc_ref, og_ref, op_ref):
        w = wc_ref[...]
        for q in range(SUB):
            og_ref[q * FEAT:(q + 1) * FEAT, :] = lax.dot_general(
                sg_ref[q:q + 1, :], w, contract,
                preferred_element_type=jnp.float32)
            op_ref[q * FEAT:(q + 1) * FEAT, :] = lax.dot_general(
                sp_ref[q:q + 1, :], w, contract,
                preferred_element_type=jnp.float32)

    return pl.pallas_call(
        body,
        grid=(OUT_GRID,),
        in_specs=[
            pl.BlockSpec((SUB, FEAT), lambda i: (i, 0)),
            pl.BlockSpec((SUB, FEAT), lambda i: (i, 0)),
            pl.BlockSpec((1, FEAT), lambda i: (0, 0)),
        ],
        out_specs=[
            pl.BlockSpec((ROW_BLK, FEAT), lambda i: (i, 0)),
            pl.BlockSpec((ROW_BLK, FEAT), lambda i: (i, 0)),
        ],
        out_shape=(jax.ShapeDtypeStruct((N, FEAT), jnp.float32),
                   jax.ShapeDtypeStruct((N, FEAT), jnp.float32)),
        compiler_params=pltpu.CompilerParams(
            dimension_semantics=("parallel",)),
    )(sg2d, sp2d, wc)


def kernel(feat_gene, edge_index_gene, feat_pro, edge_index_pro, W0, W1, W2):
    fg = feat_gene.reshape(N)
    fp = feat_pro.reshape(N)
    wc = _tc_wc(W0, W1, W2)
    sg, sp = _sc_spmm3(fg, edge_index_gene, fp, edge_index_pro)
    return _tc_outer(sg.reshape(N_PAD // FEAT, FEAT),
                     sp.reshape(N_PAD // FEAT, FEAT), wc)
